# Initial kernel scaffold; baseline (speedup 1.0000x reference)
import jax
import jax.numpy as jnp
from jax.experimental import pallas as pl


def kernel(X, edge_index, edge_weight, H, w_x_z, w_q_z, b_z, w_x_r, w_q_r, b_r, w_x_h, w_q_h, b_h):
    raise NotImplementedError("write your pallas kernel here")


# speedup vs baseline: 5.7320x; 5.7320x over previous
import functools

import jax
import jax.numpy as jnp
from jax import lax
from jax.experimental import pallas as pl
from jax.experimental.pallas import tpu as pltpu
from jax.experimental.pallas import tpu_sc as plsc

_N = 24
_E = 384
_L = 16
_NCHUNK = _E // _L
_W2 = _N * _N


def _sc_body(src_hbm, dst_hbm, ew_hbm, w_hbm, src_v, dst_v, ew_v, w_v):
    cid = lax.axis_index("c")
    sid = lax.axis_index("s")

    @pl.when(jnp.logical_and(cid == 0, sid == 0))
    def _():
        pltpu.sync_copy(src_hbm, src_v)
        pltpu.sync_copy(dst_hbm, dst_v)
        pltpu.sync_copy(ew_hbm, ew_v)

        zeros = jnp.zeros((_L,), jnp.float32)
        for i in range(_W2 // _L):
            w_v[pl.ds(i * _L, _L)] = zeros

        lane = lax.iota(jnp.int32, _L)
        for c in range(_NCHUNK):
            s = src_v[pl.ds(c * _L, _L)]
            d = dst_v[pl.ds(c * _L, _L)]
            w = ew_v[pl.ds(c * _L, _L)]
            idx = d * _N + s
            for j in range(_L):
                plsc.addupdate_scatter(w_v, [idx], w, mask=lane == j)

        pltpu.sync_copy(w_v, w_hbm)


@functools.cache
def _get_sc_build_w():
    return pl.kernel(
        _sc_body,
        out_type=jax.ShapeDtypeStruct((_W2,), jnp.float32),
        mesh=plsc.VectorSubcoreMesh(core_axis_name="c", subcore_axis_name="s"),
        scratch_types=[
            pltpu.VMEM((_E,), jnp.int32),
            pltpu.VMEM((_E,), jnp.int32),
            pltpu.VMEM((_E,), jnp.float32),
            pltpu.VMEM((_W2,), jnp.float32),
        ],
        compiler_params=pltpu.CompilerParams(needs_layout_passes=False),
    )


def _dot_t(p, w):
    return lax.dot_general(
        p, w, (((1,), (1,)), ((), ())), preferred_element_type=jnp.float32
    )


def _tc_body(w_ref, x_ref, h_ref,
             wxz_ref, wqz_ref, bz_ref,
             wxr_ref, wqr_ref, br_ref,
             wxh_ref, wqh_ref, bh_ref,
             out_ref):
    W = w_ref[...]
    deg = jnp.sum(W, axis=1)
    dinv = jnp.where(deg > 0, lax.rsqrt(jnp.where(deg > 0, deg, 1.0)), 0.0)
    A = W * dinv[:, None] * dinv[None, :]

    X = x_ref[...]
    H = h_ref[...]
    PX = jax.nn.relu(jnp.dot(A, X, preferred_element_type=jnp.float32))
    PH = jax.nn.relu(jnp.dot(A, H, preferred_element_type=jnp.float32))

    Z = jax.nn.sigmoid(_dot_t(PX, wxz_ref[...]) + _dot_t(PH, wqz_ref[...])
                       + bz_ref[...])
    R = jax.nn.sigmoid(_dot_t(PX, wxr_ref[...]) + _dot_t(PH, wqr_ref[...])
                       + br_ref[...])

    PHR = jax.nn.relu(jnp.dot(A, H * R, preferred_element_type=jnp.float32))
    Ht = jnp.tanh(_dot_t(PX, wxh_ref[...]) + _dot_t(PHR, wqh_ref[...])
                  + bh_ref[...])

    out_ref[...] = Z * Ht + (1.0 - Z) * H


_tc_gru = pl.pallas_call(
    _tc_body,
    out_shape=jax.ShapeDtypeStruct((_N, 512), jnp.float32),
)


@jax.jit
def kernel(X, edge_index, edge_weight, H,
           w_x_z, w_q_z, b_z,
           w_x_r, w_q_r, b_r,
           w_x_h, w_q_h, b_h):
    src = edge_index[0]
    dst = edge_index[1]
    w_flat = _get_sc_build_w()(src, dst, edge_weight)
    W = jnp.reshape(w_flat, (_N, _N))
    return _tc_gru(W, X, H,
                   w_x_z, w_q_z, b_z,
                   w_x_r, w_q_r, b_r,
                   w_x_h, w_q_h, b_h)


# speedup vs baseline: 6.5024x; 1.1344x over previous
import functools

import jax
import jax.numpy as jnp
from jax import lax
from jax.experimental import pallas as pl
from jax.experimental.pallas import tpu as pltpu
from jax.experimental.pallas import tpu_sc as plsc

_N = 24
_E = 384
_L = 16
_NCHUNK = _E // _L
_W2 = _N * _N


def _sc_body(src_hbm, dst_hbm, ew_hbm, w_hbm,
             src_v, dst_v, ew_v, w_v, sem0, sem1, sem2):
    c0 = pltpu.async_copy(src_hbm, src_v, sem0)
    c1 = pltpu.async_copy(dst_hbm, dst_v, sem1)
    c2 = pltpu.async_copy(ew_hbm, ew_v, sem2)

    zeros = jnp.zeros((_L,), jnp.float32)
    for r in range(_N):
        w_v[r, pl.ds(0, _L)] = zeros
        w_v[r, pl.ds(_N - _L, _L)] = zeros

    c0.wait()
    c1.wait()
    c2.wait()

    lane = lax.iota(jnp.int32, _L)
    for c in range(_NCHUNK):
        s = src_v[pl.ds(c * _L, _L)]
        d = dst_v[pl.ds(c * _L, _L)]
        w = ew_v[pl.ds(c * _L, _L)]
        for j in range(_L):
            plsc.addupdate_scatter(w_v, [d, s], w, mask=lane == j)

    pltpu.sync_copy(w_v, w_hbm)


@functools.cache
def _get_sc_build_w():
    return pl.kernel(
        _sc_body,
        out_type=jax.ShapeDtypeStruct((_N, _N), jnp.float32),
        mesh=plsc.VectorSubcoreMesh(core_axis_name="c", subcore_axis_name="s",
                                    num_cores=1, num_subcores=1),
        scratch_types=[
            pltpu.VMEM((_E,), jnp.int32),
            pltpu.VMEM((_E,), jnp.int32),
            pltpu.VMEM((_E,), jnp.float32),
            pltpu.VMEM((_N, _N), jnp.float32),
            pltpu.SemaphoreType.DMA,
            pltpu.SemaphoreType.DMA,
            pltpu.SemaphoreType.DMA,
        ],
        compiler_params=pltpu.CompilerParams(needs_layout_passes=False),
    )


def _dot_t(p, w):
    return lax.dot_general(
        p, w, (((1,), (1,)), ((), ())), preferred_element_type=jnp.float32
    )


def _tc_body(w_ref, x_ref, h_ref,
             wxz_ref, wqz_ref, bz_ref,
             wxr_ref, wqr_ref, br_ref,
             wxh_ref, wqh_ref, bh_ref,
             out_ref):
    W = w_ref[...]
    deg = jnp.sum(W, axis=1)
    dinv = jnp.where(deg > 0, lax.rsqrt(jnp.where(deg > 0, deg, 1.0)), 0.0)
    A = W * dinv[:, None] * dinv[None, :]

    X = x_ref[...]
    H = h_ref[...]
    PX = jax.nn.relu(jnp.dot(A, X, preferred_element_type=jnp.float32))
    PH = jax.nn.relu(jnp.dot(A, H, preferred_element_type=jnp.float32))

    Z = jax.nn.sigmoid(_dot_t(PX, wxz_ref[...]) + _dot_t(PH, wqz_ref[...])
                       + bz_ref[...])
    R = jax.nn.sigmoid(_dot_t(PX, wxr_ref[...]) + _dot_t(PH, wqr_ref[...])
                       + br_ref[...])

    PHR = jax.nn.relu(jnp.dot(A, H * R, preferred_element_type=jnp.float32))
    Ht = jnp.tanh(_dot_t(PX, wxh_ref[...]) + _dot_t(PHR, wqh_ref[...])
                  + bh_ref[...])

    out_ref[...] = Z * Ht + (1.0 - Z) * H


_tc_gru = pl.pallas_call(
    _tc_body,
    out_shape=jax.ShapeDtypeStruct((_N, 512), jnp.float32),
)


@jax.jit
def kernel(X, edge_index, edge_weight, H,
           w_x_z, w_q_z, b_z,
           w_x_r, w_q_r, b_r,
           w_x_h, w_q_h, b_h):
    src = edge_index[0]
    dst = edge_index[1]
    W = _get_sc_build_w()(src, dst, edge_weight)
    return _tc_gru(W, X, H,
                   w_x_z, w_q_z, b_z,
                   w_x_r, w_q_r, b_r,
                   w_x_h, w_q_h, b_h)


# speedup vs baseline: 6.5077x; 1.0008x over previous
import functools

import jax
import jax.numpy as jnp
from jax import lax
from jax.experimental import pallas as pl
from jax.experimental.pallas import tpu as pltpu
from jax.experimental.pallas import tpu_sc as plsc

_N = 24
_E = 384
_L = 16
_NCHUNK = _E // _L
_W2 = _N * _N


def _sc_body(ei_hbm, ew_hbm, w_hbm,
             src_v, dst_v, ew_v, w_v, sem0, sem1, sem2):
    c0 = pltpu.async_copy(ei_hbm.at[0], src_v, sem0)
    c1 = pltpu.async_copy(ei_hbm.at[1], dst_v, sem1)
    c2 = pltpu.async_copy(ew_hbm, ew_v, sem2)

    zeros = jnp.zeros((_L,), jnp.float32)
    for r in range(_N):
        w_v[r, pl.ds(0, _L)] = zeros
        w_v[r, pl.ds(_N - _L, _L)] = zeros

    c0.wait()
    c1.wait()
    c2.wait()

    lane = lax.iota(jnp.int32, _L)
    for c in range(_NCHUNK):
        s = src_v[pl.ds(c * _L, _L)]
        d = dst_v[pl.ds(c * _L, _L)]
        w = ew_v[pl.ds(c * _L, _L)]
        for j in range(_L):
            plsc.addupdate_scatter(w_v, [d, s], w, mask=lane == j)

    pltpu.sync_copy(w_v, w_hbm)


@functools.cache
def _get_sc_build_w():
    return pl.kernel(
        _sc_body,
        out_type=jax.ShapeDtypeStruct((_N, _N), jnp.float32),
        mesh=plsc.VectorSubcoreMesh(core_axis_name="c", subcore_axis_name="s",
                                    num_cores=1, num_subcores=1),
        scratch_types=[
            pltpu.VMEM((_E,), jnp.int32),
            pltpu.VMEM((_E,), jnp.int32),
            pltpu.VMEM((_E,), jnp.float32),
            pltpu.VMEM((_N, _N), jnp.float32),
            pltpu.SemaphoreType.DMA,
            pltpu.SemaphoreType.DMA,
            pltpu.SemaphoreType.DMA,
        ],
        compiler_params=pltpu.CompilerParams(needs_layout_passes=False),
    )


def _dot_t(p, w):
    return lax.dot_general(
        p, w, (((1,), (1,)), ((), ())), preferred_element_type=jnp.float32
    )


def _tc_body(w_ref, x_ref, h_ref,
             wxz_ref, wqz_ref, bz_ref,
             wxr_ref, wqr_ref, br_ref,
             wxh_ref, wqh_ref, bh_ref,
             out_ref):
    W = w_ref[...]
    deg = jnp.sum(W, axis=1)
    dinv = jnp.where(deg > 0, lax.rsqrt(jnp.where(deg > 0, deg, 1.0)), 0.0)
    A = W * dinv[:, None] * dinv[None, :]

    X = x_ref[...]
    H = h_ref[...]
    PX = jax.nn.relu(jnp.dot(A, X, preferred_element_type=jnp.float32))
    PH = jax.nn.relu(jnp.dot(A, H, preferred_element_type=jnp.float32))

    Z = jax.nn.sigmoid(_dot_t(PX, wxz_ref[...]) + _dot_t(PH, wqz_ref[...])
                       + bz_ref[...])
    R = jax.nn.sigmoid(_dot_t(PX, wxr_ref[...]) + _dot_t(PH, wqr_ref[...])
                       + br_ref[...])

    PHR = jax.nn.relu(jnp.dot(A, H * R, preferred_element_type=jnp.float32))
    Ht = jnp.tanh(_dot_t(PX, wxh_ref[...]) + _dot_t(PHR, wqh_ref[...])
                  + bh_ref[...])

    out_ref[...] = Z * Ht + (1.0 - Z) * H


_tc_gru = pl.pallas_call(
    _tc_body,
    out_shape=jax.ShapeDtypeStruct((_N, 512), jnp.float32),
)


@jax.jit
def kernel(X, edge_index, edge_weight, H,
           w_x_z, w_q_z, b_z,
           w_x_r, w_q_r, b_r,
           w_x_h, w_q_h, b_h):
    W = _get_sc_build_w()(edge_index, edge_weight)
    return _tc_gru(W, X, H,
                   w_x_z, w_q_z, b_z,
                   w_x_r, w_q_r, b_r,
                   w_x_h, w_q_h, b_h)
